# Initial kernel scaffold; baseline (speedup 1.0000x reference)
import jax
import jax.numpy as jnp
from jax.experimental import pallas as pl


def kernel(adj_t, edge_types, emb, basis1, comp1, root1, bias1, basis2, comp2, root2, bias2):
    raise NotImplementedError("write your pallas kernel here")


# speedup vs baseline: 2.6003x; 2.6003x over previous
import functools

import jax
import jax.numpy as jnp
from jax import lax
from jax.experimental import pallas as pl
from jax.experimental.pallas import tpu as pltpu
from jax.experimental.pallas import tpu_sc as plsc

N = 10000
R = 16
NB = 12
D = 128
E = 320000
NSEG = N * R
NTILES = 32
EPT = E // NTILES
CH = 80
NCH = EPT // CH
NPAD = 10240
RPT = NPAD // 16
BROWS = 64
SPT = NSEG // 16


def _keys_body(src_ref, dst_ref, ty_ref, ksrc_ref, kdst_ref):
    ksrc_ref[...] = src_ref[...] * R + ty_ref[...]
    kdst_ref[...] = dst_ref[...] * R + ty_ref[...]


def _make_keys(src, dst, ty):
    s2 = src.reshape(2500, 128)
    d2 = dst.reshape(2500, 128)
    t2 = ty.reshape(2500, 128)
    ks, kd = pl.pallas_call(
        _keys_body,
        out_shape=(jax.ShapeDtypeStruct((2500, 128), jnp.int32),
                   jax.ShapeDtypeStruct((2500, 128), jnp.int32)),
    )(s2, d2, t2)
    return ks.reshape(E), kd.reshape(E)


def _wrel_body(comp_ref, basis_ref, out_ref):
    r = pl.program_id(0)
    acc = comp_ref[r, 0] * basis_ref[0]
    for b in range(1, NB):
        acc = acc + comp_ref[r, b] * basis_ref[b]
    out_ref[...] = acc


def _make_waug(basis, comp, root):
    wrel = pl.pallas_call(
        _wrel_body,
        grid=(R,),
        in_specs=[
            pl.BlockSpec(memory_space=pltpu.SMEM),
            pl.BlockSpec((NB, D, D), lambda r: (0, 0, 0)),
        ],
        out_specs=pl.BlockSpec((D, D), lambda r: (0, r)),
        out_shape=jax.ShapeDtypeStruct((D, R * D), jnp.float32),
    )(comp, basis)
    return jnp.concatenate((wrel, root), axis=1)


def _mm_body(x_ref, w_ref, xt_ref, rp_ref):
    res = jnp.dot(x_ref[...], w_ref[...], preferred_element_type=jnp.float32)
    xt_ref[...] = res[:, :R * D]
    rp_ref[...] = res[:, R * D:]


def _matmul(x, waug):
    xt, rp = pl.pallas_call(
        _mm_body,
        grid=(10,),
        in_specs=[
            pl.BlockSpec((1000, D), lambda i: (i, 0)),
            pl.BlockSpec((D, R * D + D), lambda i: (0, 0)),
        ],
        out_specs=(pl.BlockSpec((1000, R * D), lambda i: (i, 0)),
                   pl.BlockSpec((1000, D), lambda i: (i, 0))),
        out_shape=(jax.ShapeDtypeStruct((N, R * D), jnp.float32),
                   jax.ShapeDtypeStruct((N, D), jnp.float32)),
    )(x, waug)
    return xt, rp


def _inv_body(c0_ref, c1_ref, out_ref):
    c = c0_ref[...] + c1_ref[...]
    out_ref[...] = 1.0 / jnp.maximum(c, 1.0)


def _make_invw(cparts):
    c0 = cparts[0].reshape(1250, 128)
    c1 = cparts[1].reshape(1250, 128)
    inv = pl.pallas_call(
        _inv_body,
        out_shape=jax.ShapeDtypeStruct((1250, 128), jnp.float32),
    )(c0, c1)
    return inv.reshape(NSEG)


def _comb_body(p0_ref, p1_ref, rp_ref, b_ref, out_ref):
    s = p0_ref[...] + p1_ref[...] + rp_ref[...] + b_ref[...]
    out_ref[...] = jnp.maximum(s, 0.0)


def _combine(p0, p1, rp, bias):
    return pl.pallas_call(
        _comb_body,
        grid=(10,),
        in_specs=[
            pl.BlockSpec((1000, D), lambda i: (i, 0)),
            pl.BlockSpec((1000, D), lambda i: (i, 0)),
            pl.BlockSpec((1000, D), lambda i: (i, 0)),
            pl.BlockSpec((1, D), lambda i: (0, 0)),
        ],
        out_specs=pl.BlockSpec((1000, D), lambda i: (i, 0)),
        out_shape=jax.ShapeDtypeStruct((N, D), jnp.float32),
    )(p0, p1, rp, bias.reshape(1, D))


_MESH = plsc.VectorSubcoreMesh(core_axis_name="c", subcore_axis_name="s")


@functools.partial(
    pl.kernel,
    out_type=jax.ShapeDtypeStruct((2 * NSEG,), jnp.float32),
    mesh=_MESH,
    scratch_types=[
        pltpu.VMEM((CH,), jnp.int32),
        pltpu.VMEM((CH,), jnp.float32),
        pltpu.VMEM((SPT,), jnp.float32),
        pltpu.VMEM_SHARED((NSEG,), jnp.float32),
    ],
)
def _sc_counts(kdst_hbm, out_hbm, kd_v, ones_v, cbuf, caccum):
    c = lax.axis_index("c")
    s = lax.axis_index("s")
    zro = jnp.zeros((16,), jnp.float32)

    def zrow(i, _):
        cbuf[pl.ds(i * 16, 16)] = zro
        return 0
    lax.fori_loop(0, SPT // 16, zrow, 0)
    for g in range(CH // 16):
        ones_v[pl.ds(g * 16, 16)] = jnp.ones((16,), jnp.float32)
    pltpu.sync_copy(cbuf, caccum.at[pl.ds(s * SPT, SPT)])
    plsc.subcore_barrier()

    base0 = (c * 16 + s) * EPT

    def chunk(it, _):
        pltpu.sync_copy(kdst_hbm.at[pl.ds(base0 + it * CH, CH)], kd_v)
        pltpu.sync_copy(ones_v, caccum.at[kd_v], add=True)
        return 0
    lax.fori_loop(0, NCH, chunk, 0)
    plsc.subcore_barrier()
    pltpu.sync_copy(caccum.at[pl.ds(s * SPT, SPT)], cbuf)
    pltpu.sync_copy(cbuf, out_hbm.at[pl.ds(c * NSEG + s * SPT, SPT)])


@functools.partial(
    pl.kernel,
    out_type=jax.ShapeDtypeStruct((2, NPAD, D), jnp.float32),
    mesh=_MESH,
    scratch_types=[
        pltpu.VMEM((CH,), jnp.int32),
        pltpu.VMEM((CH,), jnp.int32),
        pltpu.VMEM((CH,), jnp.int32),
        pltpu.VMEM((CH,), jnp.float32),
        pltpu.VMEM((CH, D), jnp.float32),
        pltpu.VMEM((BROWS, D), jnp.float32),
        pltpu.VMEM((SPT,), jnp.float32),
        pltpu.VMEM_SHARED((NSEG,), jnp.float32),
        pltpu.VMEM_SHARED((NPAD, D), jnp.float32),
        pltpu.SemaphoreType.DMA,
        pltpu.SemaphoreType.DMA,
    ],
)
def _sc_edges(ksrc_hbm, kdst_hbm, xt_hbm, invw_hbm, out_hbm,
              ks_v, kd_v, dst_v, w_v, rows_v, bbuf, tstage, tblspm, accum,
              sem1, sem2):
    c = lax.axis_index("c")
    s = lax.axis_index("s")
    zro = jnp.zeros((16,), jnp.float32)

    def zrow(i, _):
        for j in range(D // 16):
            bbuf[i, pl.ds(j * 16, 16)] = zro
        return 0
    lax.fori_loop(0, BROWS, zrow, 0)
    for k in range(RPT // BROWS):
        pltpu.sync_copy(bbuf, accum.at[pl.ds(s * RPT + k * BROWS, BROWS)])
    pltpu.sync_copy(invw_hbm.at[pl.ds(s * SPT, SPT)], tstage)
    pltpu.sync_copy(tstage, tblspm.at[pl.ds(s * SPT, SPT)])
    plsc.subcore_barrier()

    base0 = (c * 16 + s) * EPT

    def chunk(it, _):
        base = base0 + it * CH
        pltpu.sync_copy(ksrc_hbm.at[pl.ds(base, CH)], ks_v)
        pltpu.sync_copy(kdst_hbm.at[pl.ds(base, CH)], kd_v)
        for g in range(CH // 16):
            dst_v[pl.ds(g * 16, 16)] = lax.shift_right_logical(
                kd_v[pl.ds(g * 16, 16)], 4)
        cp1 = pltpu.async_copy(xt_hbm.at[ks_v], rows_v, sem1)
        cp2 = pltpu.async_copy(tblspm.at[kd_v], w_v, sem2)
        cp1.wait()
        cp2.wait()
        for g in range(CH // 16):
            w16 = w_v[pl.ds(g * 16, 16)]
            for l in range(16):
                wi = w16[l]
                row = g * 16 + l
                for j in range(D // 16):
                    rows_v[row, pl.ds(j * 16, 16)] = (
                        rows_v[row, pl.ds(j * 16, 16)] * wi)
        pltpu.sync_copy(rows_v, accum.at[dst_v], add=True)
        return 0
    lax.fori_loop(0, NCH, chunk, 0)
    plsc.subcore_barrier()
    for k in range(RPT // BROWS):
        pltpu.sync_copy(accum.at[pl.ds(s * RPT + k * BROWS, BROWS)], bbuf)
        pltpu.sync_copy(bbuf, out_hbm.at[c, pl.ds(s * RPT + k * BROWS, BROWS)])


def kernel(adj_t, edge_types, emb, basis1, comp1, root1, bias1,
           basis2, comp2, root2, bias2):
    src = adj_t[0]
    dst = adj_t[1]
    ksrc, kdst = _make_keys(src, dst, edge_types)

    waug1 = _make_waug(basis1, comp1, root1)
    waug2 = _make_waug(basis2, comp2, root2)

    cparts = _sc_counts(kdst).reshape(2, NSEG)
    invw = _make_invw(cparts)

    xt1, rp1 = _matmul(emb, waug1)
    sc1 = _sc_edges(ksrc, kdst, xt1.reshape(NSEG, D), invw)
    x1 = _combine(sc1[0, :N], sc1[1, :N], rp1, bias1)

    xt2, rp2 = _matmul(x1, waug2)
    sc2 = _sc_edges(ksrc, kdst, xt2.reshape(NSEG, D), invw)
    x2 = _combine(sc2[0, :N], sc2[1, :N], rp2, bias2)

    return jnp.concatenate((x2, x1, emb), axis=1)
